# baseline (speedup 1.0000x reference)
import jax
import jax.numpy as jnp
from jax.experimental import pallas as pl
from jax.experimental.pallas import tpu as pltpu

_MAX_SHIFT_BANDS = 2.0


def _smile_kernel(x_ref, shift_ref, out_ref, xp_ref, w_ref):
    x = x_ref[0]
    shift = shift_ref[0]
    Bh, Hblk, W = x.shape

    xb = x.astype(jnp.bfloat16)
    xp_ref[3:3 + Bh] = xb
    xp_ref[0:3] = xb[0:3]
    xp_ref[3 + Bh:] = xb[Bh - 3:Bh]

    ci = jax.lax.broadcasted_iota(jnp.int32, shift.shape, 0)
    c = ci.astype(jnp.float32)
    s = jnp.clip(shift, -_MAX_SHIFT_BANDS, _MAX_SHIFT_BANDS)
    shifted = jnp.clip(c + s, 0.0, Bh - 1.0)
    pix = shifted * (float(Bh) / float(Bh - 1)) - 0.5
    pix = jnp.clip(pix, 0.0, Bh - 1.0)

    for k in range(7):
        wd = jax.nn.relu(1.0 - jnp.abs(pix - (c + (k - 3))))
        w_ref[k] = jnp.broadcast_to(
            wd[:, None, :], (Bh, 16, W)).astype(jnp.bfloat16)

    def _hs_body(hs, _):
        h0 = hs * 16
        for t in range(Bh // 8):
            b0 = 8 * t
            acc = (w_ref[0, b0:b0 + 8]
                   * xp_ref[b0:b0 + 8, pl.ds(h0, 16), :])
            for k in range(1, 7):
                acc = acc + (w_ref[k, b0:b0 + 8]
                             * xp_ref[b0 + k:b0 + k + 8, pl.ds(h0, 16), :])
            out_ref[0, b0:b0 + 8, pl.ds(h0, 16), :] = acc.astype(jnp.float32)
        return 0

    jax.lax.fori_loop(0, Hblk // 16, _hs_body, 0)


def kernel(x, wavelength_shift):
    B, Bh, H, W = x.shape
    HBLK = 64
    grid = (B, H // HBLK)
    return pl.pallas_call(
        _smile_kernel,
        grid=grid,
        in_specs=[
            pl.BlockSpec((1, Bh, HBLK, W), lambda b, h: (b, 0, h, 0)),
            pl.BlockSpec((1, Bh, W), lambda b, h: (b, 0, 0)),
        ],
        out_specs=pl.BlockSpec((1, Bh, HBLK, W), lambda b, h: (b, 0, h, 0)),
        out_shape=jax.ShapeDtypeStruct((B, Bh, H, W), x.dtype),
        scratch_shapes=[
            pltpu.VMEM((Bh + 6, HBLK, W), jnp.bfloat16),
            pltpu.VMEM((7, Bh, 16, W), jnp.bfloat16),
        ],
        compiler_params=pltpu.CompilerParams(
            dimension_semantics=("arbitrary", "arbitrary")),
    )(x, wavelength_shift)


# speedup vs baseline: 1.0241x; 1.0241x over previous
import jax
import jax.numpy as jnp
from jax.experimental import pallas as pl
from jax.experimental.pallas import tpu as pltpu

_MAX_SHIFT_BANDS = 2.0


def _smile_kernel(x_ref, shift_ref, out_ref, xp_ref, w_ref):
    x = x_ref[0]
    shift = shift_ref[0]
    Bh, Hblk, W = x.shape

    xb = x.astype(jnp.bfloat16)
    xp_ref[3:3 + Bh] = xb
    xp_ref[0:3] = xb[0:3]
    xp_ref[3 + Bh:] = xb[Bh - 3:Bh]

    ci = jax.lax.broadcasted_iota(jnp.int32, shift.shape, 0)
    c = ci.astype(jnp.float32)
    s = jnp.clip(shift, -_MAX_SHIFT_BANDS, _MAX_SHIFT_BANDS)
    shifted = jnp.clip(c + s, 0.0, Bh - 1.0)
    pix = shifted * (float(Bh) / float(Bh - 1)) - 0.5
    pix = jnp.clip(pix, 0.0, Bh - 1.0)

    for k in range(7):
        wd = jax.nn.relu(1.0 - jnp.abs(pix - (c + (k - 3))))
        w_ref[k] = jnp.broadcast_to(
            wd[:, None, :], (Bh, 16, W)).astype(jnp.bfloat16)

    for hs in range(Hblk // 16):
        h0 = hs * 16
        for t in range(Bh // 8):
            b0 = 8 * t
            acc = (w_ref[0, b0:b0 + 8]
                   * xp_ref[b0:b0 + 8, h0:h0 + 16, :])
            for k in range(1, 7):
                acc = acc + (w_ref[k, b0:b0 + 8]
                             * xp_ref[b0 + k:b0 + k + 8, h0:h0 + 16, :])
            out_ref[0, b0:b0 + 8, h0:h0 + 16, :] = acc.astype(jnp.float32)


def kernel(x, wavelength_shift):
    B, Bh, H, W = x.shape
    HBLK = 64
    grid = (B, H // HBLK)
    return pl.pallas_call(
        _smile_kernel,
        grid=grid,
        in_specs=[
            pl.BlockSpec((1, Bh, HBLK, W), lambda b, h: (b, 0, h, 0)),
            pl.BlockSpec((1, Bh, W), lambda b, h: (b, 0, 0)),
        ],
        out_specs=pl.BlockSpec((1, Bh, HBLK, W), lambda b, h: (b, 0, h, 0)),
        out_shape=jax.ShapeDtypeStruct((B, Bh, H, W), x.dtype),
        scratch_shapes=[
            pltpu.VMEM((Bh + 6, HBLK, W), jnp.bfloat16),
            pltpu.VMEM((7, Bh, 16, W), jnp.bfloat16),
        ],
        compiler_params=pltpu.CompilerParams(
            dimension_semantics=("arbitrary", "arbitrary")),
    )(x, wavelength_shift)


# speedup vs baseline: 1.0279x; 1.0037x over previous
import jax
import jax.numpy as jnp
from jax.experimental import pallas as pl
from jax.experimental.pallas import tpu as pltpu

_MAX_SHIFT_BANDS = 2.0


def _smile_kernel(x_ref, shift_ref, out_ref, xp_ref, w_ref):
    x = x_ref[0]
    shift = shift_ref[0]
    Bh, Hblk, W = x.shape

    xb = x.astype(jnp.bfloat16)
    xp_ref[3:3 + Bh] = xb
    xp_ref[0:3] = xb[0:3]
    xp_ref[3 + Bh:] = xb[Bh - 3:Bh]

    ci = jax.lax.broadcasted_iota(jnp.int32, shift.shape, 0)
    c = ci.astype(jnp.float32)
    s = jnp.clip(shift, -_MAX_SHIFT_BANDS, _MAX_SHIFT_BANDS)
    shifted = jnp.clip(c + s, 0.0, Bh - 1.0)
    pix = shifted * (float(Bh) / float(Bh - 1)) - 0.5
    pix = jnp.clip(pix, 0.0, Bh - 1.0)

    for k in range(7):
        wd = jax.nn.relu(1.0 - jnp.abs(pix - (c + (k - 3))))
        w_ref[k] = jnp.broadcast_to(
            wd[:, None, :], (Bh, 16, W)).astype(jnp.bfloat16)

    for t in range(Bh // 8):
        b0 = 8 * t
        for hs in range(Hblk // 16):
            h0 = hs * 16
            acc = (w_ref[0, b0:b0 + 8]
                   * xp_ref[b0:b0 + 8, h0:h0 + 16, :])
            for k in range(1, 7):
                acc = acc + (w_ref[k, b0:b0 + 8]
                             * xp_ref[b0 + k:b0 + k + 8, h0:h0 + 16, :])
            out_ref[0, b0:b0 + 8, h0:h0 + 16, :] = acc.astype(jnp.float32)


def kernel(x, wavelength_shift):
    B, Bh, H, W = x.shape
    HBLK = 64
    grid = (B, H // HBLK)
    return pl.pallas_call(
        _smile_kernel,
        grid=grid,
        in_specs=[
            pl.BlockSpec((1, Bh, HBLK, W), lambda b, h: (b, 0, h, 0)),
            pl.BlockSpec((1, Bh, W), lambda b, h: (b, 0, 0)),
        ],
        out_specs=pl.BlockSpec((1, Bh, HBLK, W), lambda b, h: (b, 0, h, 0)),
        out_shape=jax.ShapeDtypeStruct((B, Bh, H, W), x.dtype),
        scratch_shapes=[
            pltpu.VMEM((Bh + 6, HBLK, W), jnp.bfloat16),
            pltpu.VMEM((7, Bh, 16, W), jnp.bfloat16),
        ],
        compiler_params=pltpu.CompilerParams(
            dimension_semantics=("arbitrary", "arbitrary")),
    )(x, wavelength_shift)
